# Initial kernel scaffold; baseline (speedup 1.0000x reference)
import jax
import jax.numpy as jnp
from jax.experimental import pallas as pl


def kernel(feats, keys, idxs):
    raise NotImplementedError("write your pallas kernel here")


# speedup vs baseline: 1.9254x; 1.9254x over previous
import functools

import jax
import jax.numpy as jnp
from jax.experimental import pallas as pl
from jax.experimental.pallas import tpu as pltpu

DIM = 128
TEMP = 0.07
EPS = 1e-12


def _loss_kernel(idxs_ref, feats_ref, keys_ref, out_ref, s_ref, t_ref, fs_ref,
                 *, tm, num_steps):
    step = pl.program_id(0)

    @pl.when(step == 0)
    def _init():
        f = feats_ref[...]
        fn = jnp.sum(f * f, axis=1, keepdims=True)
        fs_ref[...] = 1.0 / (jnp.maximum(jnp.sqrt(fn), EPS) * TEMP)
        s_ref[...] = jnp.zeros_like(s_ref)
        t_ref[...] = jnp.zeros_like(t_ref)

    k = keys_ref[...]
    kn = jnp.sum(k * k, axis=1, keepdims=True)
    ks = 1.0 / jnp.maximum(jnp.sqrt(kn), EPS)
    raw = jax.lax.dot_general(feats_ref[...], k,
                              (((1,), (1,)), ((), ())),
                              preferred_element_type=jnp.float32)
    logits = raw * fs_ref[...] * ks.T
    s_ref[...] += jnp.sum(jnp.exp(logits), axis=1, keepdims=True)

    local = idxs_ref[...] - step * tm
    col = jax.lax.broadcasted_iota(jnp.int32, logits.shape, 1)
    t_ref[...] += jnp.sum(jnp.where(col == local, logits, 0.0),
                          axis=1, keepdims=True)

    @pl.when(step == num_steps - 1)
    def _fin():
        out_ref[...] = jnp.mean(jnp.log(s_ref[...]) - t_ref[...])[None, None]


def kernel(feats, keys, idxs):
    b = feats.shape[0]
    m = keys.shape[0]
    tm = 2000
    num_steps = m // tm
    idxs2 = idxs.astype(jnp.int32).reshape(b, 1)
    out = pl.pallas_call(
        functools.partial(_loss_kernel, tm=tm, num_steps=num_steps),
        grid=(num_steps,),
        in_specs=[
            pl.BlockSpec((b, 1), lambda i: (0, 0)),
            pl.BlockSpec((b, DIM), lambda i: (0, 0)),
            pl.BlockSpec((tm, DIM), lambda i: (i, 0)),
        ],
        out_specs=pl.BlockSpec((1, 1), lambda i: (0, 0)),
        out_shape=jax.ShapeDtypeStruct((1, 1), jnp.float32),
        scratch_shapes=[
            pltpu.VMEM((b, 1), jnp.float32),
            pltpu.VMEM((b, 1), jnp.float32),
            pltpu.VMEM((b, 1), jnp.float32),
        ],
    )(idxs2, feats, keys)
    return out[0, 0]


# speedup vs baseline: 2.2860x; 1.1873x over previous
import functools

import jax
import jax.numpy as jnp
from jax.experimental import pallas as pl
from jax.experimental.pallas import tpu as pltpu

DIM = 128
TEMP = 0.07
EPS = 1e-12


def _loss_kernel(idxs_ref, feats_ref, keys_ref, out_ref, s_ref, t_ref, fsc_ref,
                 *, tm, num_steps):
    step = pl.program_id(1)

    @pl.when(step == 0)
    def _init():
        f = feats_ref[...]
        fn = jnp.sum(f * f, axis=1, keepdims=True)
        fsc_ref[...] = f * (1.0 / (jnp.maximum(jnp.sqrt(fn), EPS) * TEMP))
        s_ref[...] = jnp.zeros_like(s_ref)
        t_ref[...] = jnp.zeros_like(t_ref)

    k = keys_ref[...]
    kn = jnp.sum(k * k, axis=1, keepdims=True)
    k = k * (1.0 / jnp.maximum(jnp.sqrt(kn), EPS))
    logits = jax.lax.dot_general(fsc_ref[...], k,
                                 (((1,), (1,)), ((), ())),
                                 preferred_element_type=jnp.float32)
    s_ref[...] += jnp.sum(jnp.exp(logits), axis=1, keepdims=True)

    local = idxs_ref[...] - step * tm
    col = jax.lax.broadcasted_iota(jnp.int32, logits.shape, 1)
    t_ref[...] += jnp.sum(jnp.where(col == local, logits, 0.0),
                          axis=1, keepdims=True)

    @pl.when(step == num_steps - 1)
    def _fin():
        out_ref[...] = jnp.mean(jnp.log(s_ref[...]) - t_ref[...])[None, None, None]


def kernel(feats, keys, idxs):
    b = feats.shape[0]
    m = keys.shape[0]
    tm = 2000
    tb = b // 2
    num_steps = m // tm
    idxs2 = idxs.astype(jnp.int32).reshape(b, 1)
    out = pl.pallas_call(
        functools.partial(_loss_kernel, tm=tm, num_steps=num_steps),
        grid=(b // tb, num_steps),
        in_specs=[
            pl.BlockSpec((tb, 1), lambda i, j: (i, 0)),
            pl.BlockSpec((tb, DIM), lambda i, j: (i, 0)),
            pl.BlockSpec((tm, DIM), lambda i, j: (j, 0)),
        ],
        out_specs=pl.BlockSpec((1, 1, 1), lambda i, j: (i, 0, 0)),
        out_shape=jax.ShapeDtypeStruct((b // tb, 1, 1), jnp.float32),
        scratch_shapes=[
            pltpu.VMEM((tb, 1), jnp.float32),
            pltpu.VMEM((tb, 1), jnp.float32),
            pltpu.VMEM((tb, DIM), jnp.float32),
        ],
        compiler_params=pltpu.CompilerParams(
            dimension_semantics=("parallel", "arbitrary"),
        ),
    )(idxs2, feats, keys)
    return jnp.mean(out)


# speedup vs baseline: 2.5414x; 1.1117x over previous
import functools

import jax
import jax.numpy as jnp
from jax.experimental import pallas as pl
from jax.experimental.pallas import tpu as pltpu

DIM = 128
TEMP = 0.07
EPS = 1e-12
LOG2E = 1.4426950408889634
LN2 = 0.6931471805599453


def _loss_kernel(idxs_ref, feats_ref, keys_ref, out_ref, s_ref, t_ref, fsc_ref,
                 *, tm, num_steps):
    step = pl.program_id(1)

    @pl.when(step == 0)
    def _init():
        f = feats_ref[...]
        fn = jnp.sum(f * f, axis=1, keepdims=True)
        fsc_ref[...] = f * (LOG2E / (jnp.maximum(jnp.sqrt(fn), EPS) * TEMP))
        s_ref[...] = jnp.zeros_like(s_ref)
        t_ref[...] = jnp.zeros_like(t_ref)

    k = keys_ref[...]
    kn = jnp.sum(k * k, axis=1, keepdims=True)
    k = k * (1.0 / jnp.maximum(jnp.sqrt(kn), EPS))
    logits = jax.lax.dot_general(fsc_ref[...], k,
                                 (((1,), (1,)), ((), ())),
                                 preferred_element_type=jnp.float32)
    s_ref[...] += jnp.sum(jnp.exp2(logits), axis=1, keepdims=True)

    local = idxs_ref[...] - step * tm
    col = jax.lax.broadcasted_iota(jnp.int32, logits.shape, 1)
    t_ref[...] += jnp.sum(jnp.where(col == local, logits, 0.0),
                          axis=1, keepdims=True)

    @pl.when(step == num_steps - 1)
    def _fin():
        out_ref[...] = jnp.mean(jnp.log(s_ref[...])
                                - t_ref[...] * LN2)[None, None, None]


def kernel(feats, keys, idxs):
    b = feats.shape[0]
    m = keys.shape[0]
    tm = 2000
    tb = b // 2
    num_steps = m // tm
    idxs2 = idxs.astype(jnp.int32).reshape(b, 1)
    out = pl.pallas_call(
        functools.partial(_loss_kernel, tm=tm, num_steps=num_steps),
        grid=(b // tb, num_steps),
        in_specs=[
            pl.BlockSpec((tb, 1), lambda i, j: (i, 0)),
            pl.BlockSpec((tb, DIM), lambda i, j: (i, 0)),
            pl.BlockSpec((tm, DIM), lambda i, j: (j, 0)),
        ],
        out_specs=pl.BlockSpec((1, 1, 1), lambda i, j: (i, 0, 0)),
        out_shape=jax.ShapeDtypeStruct((b // tb, 1, 1), jnp.float32),
        scratch_shapes=[
            pltpu.VMEM((tb, 1), jnp.float32),
            pltpu.VMEM((tb, 1), jnp.float32),
            pltpu.VMEM((tb, DIM), jnp.float32),
        ],
        compiler_params=pltpu.CompilerParams(
            dimension_semantics=("parallel", "arbitrary"),
        ),
    )(idxs2, feats, keys)
    return jnp.mean(out)


# speedup vs baseline: 2.9833x; 1.1739x over previous
import functools

import jax
import jax.numpy as jnp
from jax import lax
from jax.experimental import pallas as pl
from jax.experimental.pallas import tpu as pltpu
from jax.experimental.pallas import tpu_sc as plsc

DIM = 128
TEMP = 0.07
EPS = 1e-12
LOG2E = 1.4426950408889634
LN2 = 0.6931471805599453

_SC_CORES = 2
_SC_SUBCORES = 16
_SC_WORKERS = _SC_CORES * _SC_SUBCORES


def _gather_rows_sc(keys_hbm, idx_hbm, out_hbm, idx_v, rows_v, sem,
                    *, rows_per_worker):
    wid = lax.axis_index("s") * _SC_CORES + lax.axis_index("c")
    base = wid * rows_per_worker
    pltpu.sync_copy(idx_hbm.at[pl.ds(base, rows_per_worker)], idx_v)
    pltpu.async_copy(keys_hbm.at[idx_v], rows_v, sem).wait()
    pltpu.sync_copy(rows_v, out_hbm.at[pl.ds(base, rows_per_worker)])


def _gather_rows(keys, idxs):
    b = idxs.shape[0]
    rows_per_worker = b // _SC_WORKERS
    mesh = plsc.VectorSubcoreMesh(core_axis_name="c", subcore_axis_name="s")
    return pl.kernel(
        functools.partial(_gather_rows_sc, rows_per_worker=rows_per_worker),
        out_type=jax.ShapeDtypeStruct((b, DIM), jnp.float32),
        mesh=mesh,
        scratch_types=[
            pltpu.VMEM((rows_per_worker,), jnp.int32),
            pltpu.VMEM((rows_per_worker, DIM), jnp.float32),
            pltpu.SemaphoreType.DMA,
        ],
    )(keys, idxs)


def _loss_kernel(gath_ref, feats_ref, keys_ref, out_ref, s_ref, t_ref, fsc_ref,
                 *, num_steps):
    step = pl.program_id(1)

    @pl.when(step == 0)
    def _init():
        f = feats_ref[...]
        fn = jnp.sum(f * f, axis=1, keepdims=True)
        fsc = f * (LOG2E / (jnp.maximum(jnp.sqrt(fn), EPS) * TEMP))
        fsc_ref[...] = fsc
        s_ref[...] = jnp.zeros_like(s_ref)
        g = gath_ref[...]
        gn = jnp.sum(g * g, axis=1, keepdims=True)
        t_ref[...] = (jnp.sum(fsc * g, axis=1, keepdims=True)
                      / jnp.maximum(jnp.sqrt(gn), EPS))

    k = keys_ref[...]
    kn = jnp.sum(k * k, axis=1, keepdims=True)
    k = k * (1.0 / jnp.maximum(jnp.sqrt(kn), EPS))
    logits = jax.lax.dot_general(fsc_ref[...], k,
                                 (((1,), (1,)), ((), ())),
                                 preferred_element_type=jnp.float32)
    s_ref[...] += jnp.sum(jnp.exp2(logits), axis=1, keepdims=True)

    @pl.when(step == num_steps - 1)
    def _fin():
        out_ref[...] = jnp.mean(jnp.log(s_ref[...])
                                - t_ref[...] * LN2)[None, None, None]


def kernel(feats, keys, idxs):
    b = feats.shape[0]
    m = keys.shape[0]
    tm = 2000
    tb = b // 2
    num_steps = m // tm
    gathered = _gather_rows(keys, idxs.astype(jnp.int32))
    out = pl.pallas_call(
        functools.partial(_loss_kernel, num_steps=num_steps),
        grid=(b // tb, num_steps),
        in_specs=[
            pl.BlockSpec((tb, DIM), lambda i, j: (i, 0)),
            pl.BlockSpec((tb, DIM), lambda i, j: (i, 0)),
            pl.BlockSpec((tm, DIM), lambda i, j: (j, 0)),
        ],
        out_specs=pl.BlockSpec((1, 1, 1), lambda i, j: (i, 0, 0)),
        out_shape=jax.ShapeDtypeStruct((b // tb, 1, 1), jnp.float32),
        scratch_shapes=[
            pltpu.VMEM((tb, 1), jnp.float32),
            pltpu.VMEM((tb, 1), jnp.float32),
            pltpu.VMEM((tb, DIM), jnp.float32),
        ],
        compiler_params=pltpu.CompilerParams(
            dimension_semantics=("parallel", "arbitrary"),
        ),
    )(gathered, feats, keys)
    return jnp.mean(out)
